# baseline (speedup 1.0000x reference)
import jax
import jax.numpy as jnp
from jax import lax
from jax.experimental import pallas as pl
from jax.experimental.pallas import tpu as pltpu
from jax.experimental.pallas import tpu_sc as plsc

NC = 2
NS = 16
NW = NC * NS
B = 16384
D = 32
BPW = B // NW
CHUNK = 128
NCHUNK = BPW // CHUNK
GROUPS = BPW // 16


def _body(uidx_hbm, iidx_hbm, utab_hbm, itab_hbm, bias_hbm, out_hbm,
          uidx_v, iidx_v, urows_v, irows_v, bias_v, out_v, sem):
    wid = lax.axis_index("s") * NC + lax.axis_index("c")
    base = pl.multiple_of(wid * BPW, BPW)

    pltpu.sync_copy(uidx_hbm.at[pl.ds(base, BPW)], uidx_v)
    pltpu.sync_copy(iidx_hbm.at[pl.ds(base, BPW)], iidx_v)
    pltpu.sync_copy(bias_hbm, bias_v)

    copies = []
    for c in range(NCHUNK):
        sl = pl.ds(c * CHUNK, CHUNK)
        copies.append(pltpu.async_copy(
            utab_hbm.at[uidx_v.at[sl]], urows_v.at[sl], sem))
        copies.append(pltpu.async_copy(
            itab_hbm.at[iidx_v.at[sl]], irows_v.at[sl], sem))
    for cp in copies:
        cp.wait()

    bias_vec = bias_v[...]
    row16 = lax.iota(jnp.int32, 16)

    def group(g, carry):
        rows = g * 16 + row16
        acc0 = bias_vec
        acc1 = jnp.zeros((16,), jnp.float32)
        acc2 = jnp.zeros((16,), jnp.float32)
        acc3 = jnp.zeros((16,), jnp.float32)
        accs = [acc0, acc1, acc2, acc3]
        for j in range(D):
            col = jnp.full((16,), j, jnp.int32)
            u = plsc.load_gather(urows_v, [rows, col])
            v = plsc.load_gather(irows_v, [rows, col])
            accs[j % 4] = accs[j % 4] + u * v
        total = (accs[0] + accs[1]) + (accs[2] + accs[3])
        out_v[pl.ds(pl.multiple_of(g * 16, 16), 16)] = total
        return carry

    lax.fori_loop(0, GROUPS, group, 0)
    pltpu.sync_copy(out_v, out_hbm.at[pl.ds(base, BPW)])


def kernel(user_indices, item_indices, embedding_user, embedding_item, bias):
    bias16 = jnp.broadcast_to(jnp.reshape(bias, (1,)), (16,)).astype(jnp.float32)
    mesh = plsc.VectorSubcoreMesh(core_axis_name="c", subcore_axis_name="s")
    f = pl.kernel(
        _body,
        out_type=jax.ShapeDtypeStruct((B,), jnp.float32),
        mesh=mesh,
        scratch_types=[
            pltpu.VMEM((BPW,), jnp.int32),
            pltpu.VMEM((BPW,), jnp.int32),
            pltpu.VMEM((BPW, D), jnp.float32),
            pltpu.VMEM((BPW, D), jnp.float32),
            pltpu.VMEM((16,), jnp.float32),
            pltpu.VMEM((BPW,), jnp.float32),
            pltpu.SemaphoreType.DMA,
        ],
        compiler_params=pltpu.CompilerParams(
            needs_layout_passes=False, use_tc_tiling_on_sc=False),
    )
    return f(user_indices.astype(jnp.int32), item_indices.astype(jnp.int32),
             embedding_user, embedding_item, bias16)


# speedup vs baseline: 2.0559x; 2.0559x over previous
import jax
import jax.numpy as jnp
from jax import lax
from jax.experimental import pallas as pl
from jax.experimental.pallas import tpu as pltpu
from jax.experimental.pallas import tpu_sc as plsc

NC = 2
NS = 16
NW = NC * NS
B = 16384
D = 32
BPW = B // NW
W = 8


def _body(ut_hbm, it_hbm, uidx_hbm, iidx_hbm, bias_hbm, out_hbm,
          uidx_v, iidx_v, ublk, iblk, ucomp, icomp, bias_v, out_v, sem):
    wid = lax.axis_index("s") * NC + lax.axis_index("c")
    base = pl.multiple_of(wid * BPW, BPW)

    pltpu.sync_copy(uidx_hbm.at[pl.ds(base, BPW)], uidx_v)
    pltpu.sync_copy(iidx_hbm.at[pl.ds(base, BPW)], iidx_v)
    pltpu.sync_copy(bias_hbm, bias_v)

    bias_vec = bias_v[...]
    lane16 = lax.iota(jnp.int32, 16)

    def gather_body(ci, carry):
        off16 = pl.multiple_of(ci * 16, 16)
        uvec = uidx_v[pl.ds(off16, 16)]
        ivec = iidx_v[pl.ds(off16, 16)]
        for half in range(2):
            copies = []
            for k in range(W):
                el = half * W + k
                ucol = pl.multiple_of((uvec[el] >> 7) * 128, 128)
                icol = pl.multiple_of((ivec[el] >> 7) * 128, 128)
                copies.append(pltpu.async_copy(
                    ut_hbm.at[:, pl.ds(ucol, 128)], ublk.at[k], sem))
                copies.append(pltpu.async_copy(
                    it_hbm.at[:, pl.ds(icol, 128)], iblk.at[k], sem))
            for cp in copies:
                cp.wait()
            for k in range(W):
                el = half * W + k
                n = ci * 16 + el
                ulane = jnp.full((16,), uvec[el] & 127, jnp.int32)
                ilane = jnp.full((16,), ivec[el] & 127, jnp.int32)
                kfull = jnp.full((16,), k, jnp.int32)
                for h in range(2):
                    jv = h * 16 + lane16
                    uvals = plsc.load_gather(ublk, [kfull, jv, ulane])
                    ivals = plsc.load_gather(iblk, [kfull, jv, ilane])
                    dst = pl.multiple_of(n * D + h * 16, 16)
                    ucomp[pl.ds(dst, 16)] = uvals
                    icomp[pl.ds(dst, 16)] = ivals
        return carry

    lax.fori_loop(0, BPW // 16, gather_body, 0)

    def dot_body(g, carry):
        lin0 = (g * 16 + lane16) * D
        acc0 = bias_vec
        acc1 = jnp.zeros((16,), jnp.float32)
        acc2 = jnp.zeros((16,), jnp.float32)
        acc3 = jnp.zeros((16,), jnp.float32)
        accs = [acc0, acc1, acc2, acc3]
        for j in range(D):
            lin = lin0 + j
            u = plsc.load_gather(ucomp, [lin])
            v = plsc.load_gather(icomp, [lin])
            accs[j % 4] = accs[j % 4] + u * v
        total = (accs[0] + accs[1]) + (accs[2] + accs[3])
        out_v[pl.ds(pl.multiple_of(g * 16, 16), 16)] = total
        return carry

    lax.fori_loop(0, BPW // 16, dot_body, 0)
    pltpu.sync_copy(out_v, out_hbm.at[pl.ds(base, BPW)])


def kernel(user_indices, item_indices, embedding_user, embedding_item, bias):
    bias16 = jnp.broadcast_to(jnp.reshape(bias, (1,)), (16,)).astype(jnp.float32)
    mesh = plsc.VectorSubcoreMesh(core_axis_name="c", subcore_axis_name="s")
    f = pl.kernel(
        _body,
        out_type=jax.ShapeDtypeStruct((B,), jnp.float32),
        mesh=mesh,
        scratch_types=[
            pltpu.VMEM((BPW,), jnp.int32),
            pltpu.VMEM((BPW,), jnp.int32),
            pltpu.VMEM((W, D, 128), jnp.float32),
            pltpu.VMEM((W, D, 128), jnp.float32),
            pltpu.VMEM((BPW * D,), jnp.float32),
            pltpu.VMEM((BPW * D,), jnp.float32),
            pltpu.VMEM((16,), jnp.float32),
            pltpu.VMEM((BPW,), jnp.float32),
            pltpu.SemaphoreType.DMA,
        ],
        compiler_params=pltpu.CompilerParams(needs_layout_passes=False),
    )
    return f(embedding_user.T, embedding_item.T,
             user_indices.astype(jnp.int32), item_indices.astype(jnp.int32),
             bias16)


# speedup vs baseline: 2.6153x; 1.2721x over previous
import jax
import jax.numpy as jnp
from jax import lax
from jax.experimental import pallas as pl
from jax.experimental.pallas import tpu as pltpu
from jax.experimental.pallas import tpu_sc as plsc

NC = 2
NS = 16
NW = NC * NS
B = 16384
D = 32
BPW = B // NW
W = 8
CHUNK = 128


def _user_body(ut_hbm, uidx_hbm, uvals_hbm, uidx_v, ublk, ucomp, sem):
    wid = lax.axis_index("s") * NC + lax.axis_index("c")
    base = pl.multiple_of(wid * BPW, BPW)
    pltpu.sync_copy(uidx_hbm.at[pl.ds(base, BPW)], uidx_v)
    lane16 = lax.iota(jnp.int32, 16)

    def gather_body(ci, carry):
        uvec = uidx_v[pl.ds(pl.multiple_of(ci * 16, 16), 16)]
        for half in range(2):
            copies = []
            for k in range(W):
                el = half * W + k
                ucol = pl.multiple_of((uvec[el] >> 7) * 128, 128)
                copies.append(pltpu.async_copy(
                    ut_hbm.at[:, pl.ds(ucol, 128)], ublk.at[k], sem))
            for cp in copies:
                cp.wait()
            for k in range(W):
                el = half * W + k
                n = ci * 16 + el
                ulane = jnp.full((16,), uvec[el] & 127, jnp.int32)
                kfull = jnp.full((16,), k, jnp.int32)
                for h in range(2):
                    jv = h * 16 + lane16
                    uvals = plsc.load_gather(ublk, [kfull, jv, ulane])
                    ucomp[pl.ds(pl.multiple_of(n * D + h * 16, 16), 16)] = uvals
        return carry

    lax.fori_loop(0, BPW // 16, gather_body, 0)
    pltpu.sync_copy(ucomp, uvals_hbm.at[pl.ds(base * D, BPW * D)])


def _item_dot_body(itab_hbm, iidx_hbm, uvals_hbm, bias_hbm, out_hbm,
                   iidx_v, irows_v, uvals_v, bias_v, out_v, sem):
    wid = lax.axis_index("s") * NC + lax.axis_index("c")
    base = pl.multiple_of(wid * BPW, BPW)
    pltpu.sync_copy(iidx_hbm.at[pl.ds(base, BPW)], iidx_v)
    pltpu.sync_copy(uvals_hbm.at[pl.ds(base * D, BPW * D)], uvals_v)
    pltpu.sync_copy(bias_hbm, bias_v)

    copies = []
    for c in range(BPW // CHUNK):
        sl = pl.ds(c * CHUNK, CHUNK)
        copies.append(pltpu.async_copy(
            itab_hbm.at[iidx_v.at[sl]], irows_v.at[sl], sem))
    for cp in copies:
        cp.wait()

    bias_vec = bias_v[...]
    lane16 = lax.iota(jnp.int32, 16)

    def dot_body(g, carry):
        rows = g * 16 + lane16
        lin0 = rows * D
        acc0 = bias_vec
        acc1 = jnp.zeros((16,), jnp.float32)
        acc2 = jnp.zeros((16,), jnp.float32)
        acc3 = jnp.zeros((16,), jnp.float32)
        accs = [acc0, acc1, acc2, acc3]
        for j in range(D):
            col = jnp.full((16,), j, jnp.int32)
            u = plsc.load_gather(uvals_v, [lin0 + j])
            v = plsc.load_gather(irows_v, [rows, col])
            accs[j % 4] = accs[j % 4] + u * v
        total = (accs[0] + accs[1]) + (accs[2] + accs[3])
        out_v[pl.ds(pl.multiple_of(g * 16, 16), 16)] = total
        return carry

    lax.fori_loop(0, BPW // 16, dot_body, 0)
    pltpu.sync_copy(out_v, out_hbm.at[pl.ds(base, BPW)])


def kernel(user_indices, item_indices, embedding_user, embedding_item, bias):
    bias16 = jnp.broadcast_to(jnp.reshape(bias, (1,)), (16,)).astype(jnp.float32)
    uidx = user_indices.astype(jnp.int32)
    iidx = item_indices.astype(jnp.int32)
    mesh = plsc.VectorSubcoreMesh(core_axis_name="c", subcore_axis_name="s")

    f_user = pl.kernel(
        _user_body,
        out_type=jax.ShapeDtypeStruct((B * D,), jnp.float32),
        mesh=mesh,
        scratch_types=[
            pltpu.VMEM((BPW,), jnp.int32),
            pltpu.VMEM((W, D, 128), jnp.float32),
            pltpu.VMEM((BPW * D,), jnp.float32),
            pltpu.SemaphoreType.DMA,
        ],
        compiler_params=pltpu.CompilerParams(needs_layout_passes=False),
    )
    uvals = f_user(embedding_user.T, uidx)

    f_item_dot = pl.kernel(
        _item_dot_body,
        out_type=jax.ShapeDtypeStruct((B,), jnp.float32),
        mesh=mesh,
        scratch_types=[
            pltpu.VMEM((BPW,), jnp.int32),
            pltpu.VMEM((BPW, D), jnp.float32),
            pltpu.VMEM((BPW * D,), jnp.float32),
            pltpu.VMEM((16,), jnp.float32),
            pltpu.VMEM((BPW,), jnp.float32),
            pltpu.SemaphoreType.DMA,
        ],
        compiler_params=pltpu.CompilerParams(
            needs_layout_passes=False, use_tc_tiling_on_sc=False),
    )
    return f_item_dot(embedding_item, iidx, uvals, bias16)


# speedup vs baseline: 2.9684x; 1.1350x over previous
import jax
import jax.numpy as jnp
from jax import lax
from jax.experimental import pallas as pl
from jax.experimental.pallas import tpu as pltpu
from jax.experimental.pallas import tpu_sc as plsc

NC = 2
NS = 16
NW = NC * NS
B = 16384
D = 32
BPW = B // NW
W = 8
CHUNK = 128


def _user_body(ut_hbm, uidx_hbm, uvals_hbm, uidx_v, ublk, ucomp, sem_a, sem_b):
    wid = lax.axis_index("s") * NC + lax.axis_index("c")
    base = pl.multiple_of(wid * BPW, BPW)
    pltpu.sync_copy(uidx_hbm.at[pl.ds(base, BPW)], uidx_v)
    lane16 = lax.iota(jnp.int32, 16)
    sems = [sem_a, sem_b]

    def load_vec(ci):
        return uidx_v[pl.ds(pl.multiple_of(ci * 16, 16), 16)]

    def fire(uvec, half, sset):
        for k in range(W):
            el = half * W + k
            ucol = pl.multiple_of((uvec[el] >> 7) * 128, 128)
            pltpu.async_copy(
                ut_hbm.at[:, pl.ds(ucol, 128)], ublk.at[sset, k], sems[sset])

    def drain(sset):
        for k in range(W):
            pltpu.make_async_copy(
                ut_hbm.at[:, pl.ds(0, 128)], ublk.at[sset, k], sems[sset]).wait()

    def extract(uvec, ci, half, sset):
        pfull = jnp.full((16,), sset, jnp.int32)
        for k in range(W):
            el = half * W + k
            n = ci * 16 + el
            ulane = jnp.full((16,), uvec[el] & 127, jnp.int32)
            kfull = jnp.full((16,), k, jnp.int32)
            for h in range(2):
                jv = h * 16 + lane16
                uvals = plsc.load_gather(ublk, [pfull, kfull, jv, ulane])
                ucomp[pl.ds(pl.multiple_of(n * D + h * 16, 16), 16)] = uvals

    vec0 = load_vec(0)
    fire(vec0, 0, 0)
    fire(vec0, 1, 1)

    def gather_body(ci, uvec_prev):
        uvec = load_vec(ci)
        drain(0)
        extract(uvec_prev, ci - 1, 0, 0)
        fire(uvec, 0, 0)
        drain(1)
        extract(uvec_prev, ci - 1, 1, 1)
        fire(uvec, 1, 1)
        return uvec

    last = lax.fori_loop(1, BPW // 16, gather_body, vec0)
    drain(0)
    extract(last, BPW // 16 - 1, 0, 0)
    drain(1)
    extract(last, BPW // 16 - 1, 1, 1)
    pltpu.sync_copy(ucomp, uvals_hbm.at[pl.ds(base * D, BPW * D)])


def _item_dot_body(itab_hbm, iidx_hbm, uvals_hbm, bias_hbm, out_hbm,
                   iidx_v, irows_v, uvals_v, bias_v, out_v, sem):
    wid = lax.axis_index("s") * NC + lax.axis_index("c")
    base = pl.multiple_of(wid * BPW, BPW)
    pltpu.sync_copy(iidx_hbm.at[pl.ds(base, BPW)], iidx_v)
    pltpu.sync_copy(uvals_hbm.at[pl.ds(base * D, BPW * D)], uvals_v)
    pltpu.sync_copy(bias_hbm, bias_v)

    copies = []
    for c in range(BPW // CHUNK):
        sl = pl.ds(c * CHUNK, CHUNK)
        copies.append(pltpu.async_copy(
            itab_hbm.at[iidx_v.at[sl]], irows_v.at[sl], sem))
    for cp in copies:
        cp.wait()

    bias_vec = bias_v[...]
    lane16 = lax.iota(jnp.int32, 16)

    def dot_body(g, carry):
        rows = g * 16 + lane16
        lin0 = rows * D
        acc0 = bias_vec
        acc1 = jnp.zeros((16,), jnp.float32)
        acc2 = jnp.zeros((16,), jnp.float32)
        acc3 = jnp.zeros((16,), jnp.float32)
        accs = [acc0, acc1, acc2, acc3]
        for j in range(D):
            col = jnp.full((16,), j, jnp.int32)
            u = plsc.load_gather(uvals_v, [lin0 + j])
            v = plsc.load_gather(irows_v, [rows, col])
            accs[j % 4] = accs[j % 4] + u * v
        total = (accs[0] + accs[1]) + (accs[2] + accs[3])
        out_v[pl.ds(pl.multiple_of(g * 16, 16), 16)] = total
        return carry

    lax.fori_loop(0, BPW // 16, dot_body, 0)
    pltpu.sync_copy(out_v, out_hbm.at[pl.ds(base, BPW)])


def kernel(user_indices, item_indices, embedding_user, embedding_item, bias):
    bias16 = jnp.broadcast_to(jnp.reshape(bias, (1,)), (16,)).astype(jnp.float32)
    uidx = user_indices.astype(jnp.int32)
    iidx = item_indices.astype(jnp.int32)
    mesh = plsc.VectorSubcoreMesh(core_axis_name="c", subcore_axis_name="s")

    f_user = pl.kernel(
        _user_body,
        out_type=jax.ShapeDtypeStruct((B * D,), jnp.float32),
        mesh=mesh,
        scratch_types=[
            pltpu.VMEM((BPW,), jnp.int32),
            pltpu.VMEM((2, W, D, 128), jnp.float32),
            pltpu.VMEM((BPW * D,), jnp.float32),
            pltpu.SemaphoreType.DMA,
            pltpu.SemaphoreType.DMA,
        ],
        compiler_params=pltpu.CompilerParams(needs_layout_passes=False),
    )
    uvals = f_user(embedding_user.T, uidx)

    f_item_dot = pl.kernel(
        _item_dot_body,
        out_type=jax.ShapeDtypeStruct((B,), jnp.float32),
        mesh=mesh,
        scratch_types=[
            pltpu.VMEM((BPW,), jnp.int32),
            pltpu.VMEM((BPW, D), jnp.float32),
            pltpu.VMEM((BPW * D,), jnp.float32),
            pltpu.VMEM((16,), jnp.float32),
            pltpu.VMEM((BPW,), jnp.float32),
            pltpu.SemaphoreType.DMA,
        ],
        compiler_params=pltpu.CompilerParams(
            needs_layout_passes=False, use_tc_tiling_on_sc=False),
    )
    return f_item_dot(embedding_item, iidx, uvals, bias16)
